# baseline (speedup 1.0000x reference)
import functools

import jax
import jax.numpy as jnp
from jax import lax
from jax.experimental import pallas as pl
from jax.experimental.pallas import tpu as pltpu
from jax.experimental.pallas import tpu_sc as plsc

N = 10000
D = 128
H = 64
B = 64
E = 320000

NP = 10240
EP = 327680
NW = 32
EPW = EP // NW
CH = 128
NCH = EPW // CH
RPT = NP // 16
NBUF = 8
NAHEAD = 4
R = 1280
GRID = NP // R

_mesh = plsc.VectorSubcoreMesh(core_axis_name="c", subcore_axis_name="s")


@functools.partial(
    pl.kernel,
    out_type=jax.ShapeDtypeStruct((NW, NP), jnp.float32),
    mesh=_mesh,
    scratch_types=[
        pltpu.VMEM((EPW,), jnp.int32),
        pltpu.VMEM((NP,), jnp.float32),
    ],
    compiler_params=pltpu.CompilerParams(needs_layout_passes=False),
)
def _deg(dst_hbm, out_hbm, idx_v, deg_v):
    cid = lax.axis_index("c")
    sid = lax.axis_index("s")
    wid = sid * 2 + cid
    pltpu.sync_copy(dst_hbm.at[pl.ds(wid * EPW, EPW)], idx_v)

    zeros16 = jnp.zeros((16,), jnp.float32)
    ones16 = jnp.ones((16,), jnp.float32)

    def zbody(i, _):
        deg_v[pl.ds(i * 16, 16)] = zeros16
        return 0

    lax.fori_loop(0, NP // 16, zbody, 0)

    def sbody(t, _):
        idx = idx_v[pl.ds(t * 16, 16)]
        plsc.addupdate_scatter(deg_v, [idx], ones16)
        return 0

    lax.fori_loop(0, EPW // 16, sbody, 0)
    pltpu.sync_copy(deg_v, out_hbm.at[wid])


@functools.partial(
    pl.kernel,
    out_type=jax.ShapeDtypeStruct((2 * NP, H), jnp.float32),
    mesh=_mesh,
    scratch_types=[
        pltpu.VMEM((NCH, CH), jnp.int32),
        pltpu.VMEM((NCH, CH), jnp.int32),
        [pltpu.VMEM((CH, H), jnp.float32)] * NBUF,
        [pltpu.SemaphoreType.DMA] * NBUF,
        [pltpu.SemaphoreType.DMA] * NBUF,
        pltpu.VMEM_SHARED((NP, H), jnp.float32),
    ],
    compiler_params=pltpu.CompilerParams(use_tc_tiling_on_sc=False),
)
def _agg(g_hbm, src_hbm, dst_hbm, z_hbm, out_hbm,
         sidx, didx, rows, gsem, ssem, acc):
    cid = lax.axis_index("c")
    sid = lax.axis_index("s")
    wid = sid * 2 + cid

    pltpu.sync_copy(z_hbm.at[pl.ds(sid * RPT, RPT)], acc.at[pl.ds(sid * RPT, RPT)])
    pltpu.sync_copy(src_hbm.at[pl.ds(wid * NCH, NCH)], sidx)
    pltpu.sync_copy(dst_hbm.at[pl.ds(wid * NCH, NCH)], didx)
    plsc.subcore_barrier()

    def gather(c, b):
        pltpu.async_copy(g_hbm.at[sidx.at[c]], rows[b], gsem[b])

    def wait_gather(b):
        pltpu.make_async_copy(g_hbm.at[sidx.at[0]], rows[b], gsem[b]).wait()

    def scatter(c, b):
        pltpu.async_copy(rows[b], acc.at[didx.at[c]], ssem[b], add=True)

    def wait_scatter(b):
        pltpu.make_async_copy(rows[b], acc.at[didx.at[0]], ssem[b]).wait()

    for u in range(NAHEAD):
        gather(u, u)

    def body(t, _):
        for u in range(NBUF):
            c = t * NBUF + u
            wait_gather(u)
            scatter(c, u)
            c2 = c + NAHEAD
            b2 = (u + NAHEAD) % NBUF

            @pl.when(c2 < NCH)
            def _():
                @pl.when(c2 >= NBUF)
                def _():
                    wait_scatter(b2)

                gather(c2, b2)

        return 0

    lax.fori_loop(0, NCH // NBUF, body, 0)
    for u in range(NBUF):
        wait_scatter(u)
    plsc.subcore_barrier()

    def cbody(j, _):
        r0 = sid * RPT + j * CH
        pltpu.sync_copy(acc.at[pl.ds(r0, CH)], rows[0])
        pltpu.sync_copy(rows[0], out_hbm.at[pl.ds(cid * NP + r0, CH)])
        return 0

    lax.fori_loop(0, RPT // CH, cbody, 0)


def _dis_block(degp):
    return lax.rsqrt(1.0 + jnp.sum(degp, axis=0))[:, None]


def _tc1_body(degp_ref, x_ref, w1_ref, g1_ref):
    dis = _dis_block(degp_ref[...])
    h = jnp.dot(x_ref[...], w1_ref[...], preferred_element_type=jnp.float32)
    g1_ref[...] = h * dis


def _tc2_body(accp_ref, g1_ref, degp_ref, b1_ref, w2_ref, g2_ref):
    dis = _dis_block(degp_ref[...])
    a = accp_ref[0] + accp_ref[1] + g1_ref[...]
    h1 = jnp.maximum(a * dis + b1_ref[...], 0.0)
    g2_ref[...] = jnp.dot(h1, w2_ref[...], preferred_element_type=jnp.float32) * dis


def _tc3_body(accp_ref, g2_ref, degp_ref, b2_ref, batch_ref, wct_ref, bc_ref,
              out_ref, s_acc, c_acc):
    i = pl.program_id(0)

    @pl.when(i == 0)
    def _():
        s_acc[...] = jnp.zeros((B, H), jnp.float32)
        c_acc[...] = jnp.zeros((B, H), jnp.float32)

    dis = _dis_block(degp_ref[...])
    a = accp_ref[0] + accp_ref[1] + g2_ref[...]
    h = jnp.maximum(a * dis + b2_ref[...], 0.0)

    bt = batch_ref[0, 0, :]
    classes = lax.broadcasted_iota(jnp.int32, (B, R), 0)
    onehot = (classes == bt[None, :]).astype(jnp.float32)
    s_acc[...] += jnp.dot(onehot, h, preferred_element_type=jnp.float32)
    c_acc[...] += jnp.broadcast_to(
        jnp.sum(onehot, axis=1, keepdims=True), (B, H))

    @pl.when(i == GRID - 1)
    def _():
        pooled = s_acc[...] / jnp.maximum(c_acc[...], 1.0)
        logits = jnp.sum(pooled * wct_ref[...], axis=1, keepdims=True) + bc_ref[0]
        out_ref[...] = jax.nn.sigmoid(logits)


def _tc1(degp, xp, W1):
    return pl.pallas_call(
        _tc1_body,
        grid=(GRID,),
        in_specs=[
            pl.BlockSpec((NW, R), lambda i: (0, i)),
            pl.BlockSpec((R, D), lambda i: (i, 0)),
            pl.BlockSpec((D, H), lambda i: (0, 0)),
        ],
        out_specs=pl.BlockSpec((R, H), lambda i: (i, 0)),
        out_shape=jax.ShapeDtypeStruct((NP, H), jnp.float32),
    )(degp, xp, W1)


def _tc2(accp, g1, degp, b1r, W2):
    return pl.pallas_call(
        _tc2_body,
        grid=(GRID,),
        in_specs=[
            pl.BlockSpec((2, R, H), lambda i: (0, i, 0)),
            pl.BlockSpec((R, H), lambda i: (i, 0)),
            pl.BlockSpec((NW, R), lambda i: (0, i)),
            pl.BlockSpec((1, H), lambda i: (0, 0)),
            pl.BlockSpec((H, H), lambda i: (0, 0)),
        ],
        out_specs=pl.BlockSpec((R, H), lambda i: (i, 0)),
        out_shape=jax.ShapeDtypeStruct((NP, H), jnp.float32),
    )(accp, g1, degp, b1r, W2)


def _tc3(accp, g2, degp, b2r, batch3d, WcT, bc):
    return pl.pallas_call(
        _tc3_body,
        grid=(GRID,),
        in_specs=[
            pl.BlockSpec((2, R, H), lambda i: (0, i, 0)),
            pl.BlockSpec((R, H), lambda i: (i, 0)),
            pl.BlockSpec((NW, R), lambda i: (0, i)),
            pl.BlockSpec((1, H), lambda i: (0, 0)),
            pl.BlockSpec((1, 1, R), lambda i: (i, 0, 0)),
            pl.BlockSpec((1, H), lambda i: (0, 0)),
            pl.BlockSpec(memory_space=pltpu.SMEM),
        ],
        out_specs=pl.BlockSpec((B, 1), lambda i: (0, 0)),
        out_shape=jax.ShapeDtypeStruct((B, 1), jnp.float32),
        scratch_shapes=[
            pltpu.VMEM((B, H), jnp.float32),
            pltpu.VMEM((B, H), jnp.float32),
        ],
    )(accp, g2, degp, b2r, batch3d, WcT, bc)


def kernel(x, edge_index, batch, W1, b1, W2, b2, Wc, bc):
    xp = jnp.pad(x, ((0, NP - N), (0, 0)))
    pad_idx = jnp.full((EP - E,), NP - 1, jnp.int32)
    src = jnp.concatenate([edge_index[0], pad_idx])
    dst = jnp.concatenate([edge_index[1], pad_idx])
    src2d = src.reshape(EP // CH, CH)
    dst2d = dst.reshape(EP // CH, CH)
    batch3d = jnp.pad(batch, (0, NP - N), constant_values=B).reshape(GRID, 1, R)
    zrows = jnp.zeros((NP, H), jnp.float32)
    b1r = b1.reshape(1, H)
    b2r = b2.reshape(1, H)
    WcT = Wc.reshape(1, H)

    degp = _deg(dst)
    g1 = _tc1(degp, xp, W1)
    accp1 = _agg(g1, src2d, dst2d, zrows).reshape(2, NP, H)
    g2 = _tc2(accp1, g1, degp, b1r, W2)
    accp2 = _agg(g2, src2d, dst2d, zrows).reshape(2, NP, H)
    out2d = _tc3(accp2, g2, degp, b2r, batch3d, WcT, bc)
    return out2d.reshape(B)


# speedup vs baseline: 2.8782x; 2.8782x over previous
import functools

import jax
import jax.numpy as jnp
from jax import lax
from jax.experimental import pallas as pl
from jax.experimental.pallas import tpu as pltpu
from jax.experimental.pallas import tpu_sc as plsc

N = 10000
D = 128
H = 64
B = 64
E = 320000

NP = 10240
EP = 327680
NW = 32
EPW = EP // NW
CH = 128
NCH = EPW // CH
RPT = NP // 16
NBUF = 8
NAHEAD = 4
R = 1280
GRID = NP // R

_mesh = plsc.VectorSubcoreMesh(core_axis_name="c", subcore_axis_name="s")


@functools.partial(
    pl.kernel,
    out_type=jax.ShapeDtypeStruct((NW, NP), jnp.float32),
    mesh=_mesh,
    scratch_types=[
        pltpu.VMEM((EPW,), jnp.int32),
        pltpu.VMEM((NP,), jnp.float32),
    ],
    compiler_params=pltpu.CompilerParams(needs_layout_passes=False),
)
def _deg(dst_hbm, out_hbm, idx_v, deg_v):
    cid = lax.axis_index("c")
    sid = lax.axis_index("s")
    wid = sid * 2 + cid
    pltpu.sync_copy(dst_hbm.at[pl.ds(wid * EPW, EPW)], idx_v)

    zeros16 = jnp.zeros((16,), jnp.float32)
    ones16 = jnp.ones((16,), jnp.float32)

    def zbody(i, _):
        deg_v[pl.ds(i * 16, 16)] = zeros16
        return 0

    lax.fori_loop(0, NP // 16, zbody, 0)

    def sbody(t, _):
        idx = idx_v[pl.ds(t * 16, 16)]
        plsc.addupdate_scatter(deg_v, [idx], ones16)
        return 0

    lax.fori_loop(0, EPW // 16, sbody, 0)
    pltpu.sync_copy(deg_v, out_hbm.at[wid])


@functools.partial(
    pl.kernel,
    out_type=jax.ShapeDtypeStruct((2 * NP, H), jnp.bfloat16),
    mesh=_mesh,
    scratch_types=[
        pltpu.VMEM((NCH, CH), jnp.int32),
        pltpu.VMEM((NCH, CH), jnp.int32),
        [pltpu.VMEM((CH, H), jnp.bfloat16)] * NBUF,
        [pltpu.SemaphoreType.DMA] * NBUF,
        [pltpu.SemaphoreType.DMA] * NBUF,
        pltpu.VMEM_SHARED((NP, H), jnp.bfloat16),
        pltpu.VMEM_SHARED((NP, H), jnp.bfloat16),
    ],
    compiler_params=pltpu.CompilerParams(use_tc_tiling_on_sc=False),
)
def _agg(g_hbm, src_hbm, dst_hbm, z_hbm, out_hbm,
         sidx, didx, rows, gsem, ssem, acc, gsp):
    cid = lax.axis_index("c")
    sid = lax.axis_index("s")
    wid = sid * 2 + cid

    pltpu.sync_copy(z_hbm.at[pl.ds(sid * RPT, RPT)], acc.at[pl.ds(sid * RPT, RPT)])
    pltpu.sync_copy(g_hbm.at[pl.ds(sid * RPT, RPT)], gsp.at[pl.ds(sid * RPT, RPT)])
    pltpu.sync_copy(src_hbm.at[pl.ds(wid * NCH, NCH)], sidx)
    pltpu.sync_copy(dst_hbm.at[pl.ds(wid * NCH, NCH)], didx)
    plsc.subcore_barrier()

    def gather(c, b):
        pltpu.async_copy(gsp.at[sidx.at[c]], rows[b], gsem[b])

    def wait_gather(b):
        pltpu.make_async_copy(gsp.at[sidx.at[0]], rows[b], gsem[b]).wait()

    def scatter(c, b):
        pltpu.async_copy(rows[b], acc.at[didx.at[c]], ssem[b], add=True)

    def wait_scatter(b):
        pltpu.make_async_copy(rows[b], acc.at[didx.at[0]], ssem[b]).wait()

    for u in range(NAHEAD):
        gather(u, u)

    def body(t, _):
        for u in range(NBUF):
            c = t * NBUF + u
            wait_gather(u)
            scatter(c, u)
            c2 = c + NAHEAD
            b2 = (u + NAHEAD) % NBUF

            @pl.when(c2 < NCH)
            def _():
                @pl.when(c2 >= NBUF)
                def _():
                    wait_scatter(b2)

                gather(c2, b2)

        return 0

    lax.fori_loop(0, NCH // NBUF, body, 0)
    for u in range(NBUF):
        wait_scatter(u)
    plsc.subcore_barrier()

    def cbody(j, _):
        r0 = sid * RPT + j * CH
        pltpu.sync_copy(acc.at[pl.ds(r0, CH)], rows[0])
        pltpu.sync_copy(rows[0], out_hbm.at[pl.ds(cid * NP + r0, CH)])
        return 0

    lax.fori_loop(0, RPT // CH, cbody, 0)


def _dis_block(degp):
    return lax.rsqrt(1.0 + jnp.sum(degp, axis=0))[:, None]


def _tc1_body(degp_ref, x_ref, w1_ref, g1_ref):
    dis = _dis_block(degp_ref[...])
    h = jnp.dot(x_ref[...], w1_ref[...], preferred_element_type=jnp.float32)
    g1_ref[...] = (h * dis).astype(jnp.bfloat16)


def _tc2_body(accp_ref, g1_ref, degp_ref, b1_ref, w2_ref, g2_ref):
    dis = _dis_block(degp_ref[...])
    a = (accp_ref[0].astype(jnp.float32) + accp_ref[1].astype(jnp.float32)
         + g1_ref[...].astype(jnp.float32))
    h1 = jnp.maximum(a * dis + b1_ref[...], 0.0)
    g2_ref[...] = (jnp.dot(h1, w2_ref[...], preferred_element_type=jnp.float32)
                   * dis).astype(jnp.bfloat16)


def _tc3_body(accp_ref, g2_ref, degp_ref, b2_ref, batch_ref, wct_ref, bc_ref,
              out_ref, s_acc, c_acc):
    i = pl.program_id(0)

    @pl.when(i == 0)
    def _():
        s_acc[...] = jnp.zeros((B, H), jnp.float32)
        c_acc[...] = jnp.zeros((B, H), jnp.float32)

    dis = _dis_block(degp_ref[...])
    a = (accp_ref[0].astype(jnp.float32) + accp_ref[1].astype(jnp.float32)
         + g2_ref[...].astype(jnp.float32))
    h = jnp.maximum(a * dis + b2_ref[...], 0.0)

    bt = batch_ref[0, 0, :]
    classes = lax.broadcasted_iota(jnp.int32, (B, R), 0)
    onehot = (classes == bt[None, :]).astype(jnp.float32)
    s_acc[...] += jnp.dot(onehot, h, preferred_element_type=jnp.float32)
    c_acc[...] += jnp.broadcast_to(
        jnp.sum(onehot, axis=1, keepdims=True), (B, H))

    @pl.when(i == GRID - 1)
    def _():
        pooled = s_acc[...] / jnp.maximum(c_acc[...], 1.0)
        logits = jnp.sum(pooled * wct_ref[...], axis=1, keepdims=True) + bc_ref[0]
        out_ref[...] = jax.nn.sigmoid(logits)


def _tc1(degp, xp, W1):
    return pl.pallas_call(
        _tc1_body,
        grid=(GRID,),
        in_specs=[
            pl.BlockSpec((NW, R), lambda i: (0, i)),
            pl.BlockSpec((R, D), lambda i: (i, 0)),
            pl.BlockSpec((D, H), lambda i: (0, 0)),
        ],
        out_specs=pl.BlockSpec((R, H), lambda i: (i, 0)),
        out_shape=jax.ShapeDtypeStruct((NP, H), jnp.bfloat16),
    )(degp, xp, W1)


def _tc2(accp, g1, degp, b1r, W2):
    return pl.pallas_call(
        _tc2_body,
        grid=(GRID,),
        in_specs=[
            pl.BlockSpec((2, R, H), lambda i: (0, i, 0)),
            pl.BlockSpec((R, H), lambda i: (i, 0)),
            pl.BlockSpec((NW, R), lambda i: (0, i)),
            pl.BlockSpec((1, H), lambda i: (0, 0)),
            pl.BlockSpec((H, H), lambda i: (0, 0)),
        ],
        out_specs=pl.BlockSpec((R, H), lambda i: (i, 0)),
        out_shape=jax.ShapeDtypeStruct((NP, H), jnp.bfloat16),
    )(accp, g1, degp, b1r, W2)


def _tc3(accp, g2, degp, b2r, batch3d, WcT, bc):
    return pl.pallas_call(
        _tc3_body,
        grid=(GRID,),
        in_specs=[
            pl.BlockSpec((2, R, H), lambda i: (0, i, 0)),
            pl.BlockSpec((R, H), lambda i: (i, 0)),
            pl.BlockSpec((NW, R), lambda i: (0, i)),
            pl.BlockSpec((1, H), lambda i: (0, 0)),
            pl.BlockSpec((1, 1, R), lambda i: (i, 0, 0)),
            pl.BlockSpec((1, H), lambda i: (0, 0)),
            pl.BlockSpec(memory_space=pltpu.SMEM),
        ],
        out_specs=pl.BlockSpec((B, 1), lambda i: (0, 0)),
        out_shape=jax.ShapeDtypeStruct((B, 1), jnp.float32),
        scratch_shapes=[
            pltpu.VMEM((B, H), jnp.float32),
            pltpu.VMEM((B, H), jnp.float32),
        ],
    )(accp, g2, degp, b2r, batch3d, WcT, bc)


def kernel(x, edge_index, batch, W1, b1, W2, b2, Wc, bc):
    xp = jnp.pad(x, ((0, NP - N), (0, 0)))
    pad_idx = jnp.full((EP - E,), NP - 1, jnp.int32)
    src = jnp.concatenate([edge_index[0], pad_idx])
    dst = jnp.concatenate([edge_index[1], pad_idx])
    src2d = src.reshape(EP // CH, CH)
    dst2d = dst.reshape(EP // CH, CH)
    batch3d = jnp.pad(batch, (0, NP - N), constant_values=B).reshape(GRID, 1, R)
    zrows = jnp.zeros((NP, H), jnp.bfloat16)
    b1r = b1.reshape(1, H)
    b2r = b2.reshape(1, H)
    WcT = Wc.reshape(1, H)

    degp = _deg(dst)
    g1 = _tc1(degp, xp, W1)
    accp1 = _agg(g1, src2d, dst2d, zrows).reshape(2, NP, H)
    g2 = _tc2(accp1, g1, degp, b1r, W2)
    accp2 = _agg(g2, src2d, dst2d, zrows).reshape(2, NP, H)
    out2d = _tc3(accp2, g2, degp, b2r, batch3d, WcT, bc)
    return out2d.reshape(B)


# speedup vs baseline: 2.9071x; 1.0101x over previous
import functools

import jax
import jax.numpy as jnp
from jax import lax
from jax.experimental import pallas as pl
from jax.experimental.pallas import tpu as pltpu
from jax.experimental.pallas import tpu_sc as plsc

N = 10000
D = 128
H = 64
B = 64
E = 320000

NP = 10240
EP = 327680
NW = 32
EPW = EP // NW
CH = 128
NCH = EPW // CH
RPT = NP // 16
NBUF = 8
NAHEAD = 4
R = 1280
GRID = NP // R

_mesh = plsc.VectorSubcoreMesh(core_axis_name="c", subcore_axis_name="s")


@functools.partial(
    pl.kernel,
    out_type=jax.ShapeDtypeStruct((NW, NP), jnp.float32),
    mesh=_mesh,
    scratch_types=[
        pltpu.VMEM((EPW,), jnp.int32),
        pltpu.VMEM((NP,), jnp.float32),
    ],
    compiler_params=pltpu.CompilerParams(needs_layout_passes=False),
)
def _deg(dst_hbm, out_hbm, idx_v, deg_v):
    cid = lax.axis_index("c")
    sid = lax.axis_index("s")
    wid = sid * 2 + cid
    pltpu.sync_copy(dst_hbm.at[pl.ds(wid * EPW, EPW)], idx_v)

    zeros16 = jnp.zeros((16,), jnp.float32)
    ones16 = jnp.ones((16,), jnp.float32)

    def zbody(i, _):
        for u in range(4):
            deg_v[pl.ds((i * 4 + u) * 16, 16)] = zeros16
        return 0

    lax.fori_loop(0, NP // 64, zbody, 0)

    def sbody(t, _):
        for u in range(4):
            idx = idx_v[pl.ds((t * 4 + u) * 16, 16)]
            plsc.addupdate_scatter(deg_v, [idx], ones16)
        return 0

    lax.fori_loop(0, EPW // 64, sbody, 0)
    pltpu.sync_copy(deg_v, out_hbm.at[wid])


@functools.partial(
    pl.kernel,
    out_type=jax.ShapeDtypeStruct((2 * NP, H), jnp.bfloat16),
    mesh=_mesh,
    scratch_types=[
        pltpu.VMEM((NCH, CH), jnp.int32),
        pltpu.VMEM((NCH, CH), jnp.int32),
        [pltpu.VMEM((CH, H), jnp.bfloat16)] * NBUF,
        [pltpu.SemaphoreType.DMA] * NBUF,
        [pltpu.SemaphoreType.DMA] * NBUF,
        pltpu.VMEM_SHARED((NP, H), jnp.bfloat16),
        pltpu.VMEM_SHARED((NP, H), jnp.bfloat16),
    ],
    compiler_params=pltpu.CompilerParams(use_tc_tiling_on_sc=False),
)
def _agg(g_hbm, src_hbm, dst_hbm, z_hbm, out_hbm,
         sidx, didx, rows, gsem, ssem, acc, gsp):
    cid = lax.axis_index("c")
    sid = lax.axis_index("s")
    wid = sid * 2 + cid

    pltpu.sync_copy(z_hbm.at[pl.ds(sid * RPT, RPT)], acc.at[pl.ds(sid * RPT, RPT)])
    pltpu.sync_copy(g_hbm.at[pl.ds(sid * RPT, RPT)], gsp.at[pl.ds(sid * RPT, RPT)])
    pltpu.sync_copy(src_hbm.at[pl.ds(wid * NCH, NCH)], sidx)
    pltpu.sync_copy(dst_hbm.at[pl.ds(wid * NCH, NCH)], didx)
    plsc.subcore_barrier()

    def gather(c, b):
        pltpu.async_copy(gsp.at[sidx.at[c]], rows[b], gsem[b])

    def wait_gather(b):
        pltpu.make_async_copy(gsp.at[sidx.at[0]], rows[b], gsem[b]).wait()

    def scatter(c, b):
        pltpu.async_copy(rows[b], acc.at[didx.at[c]], ssem[b], add=True)

    def wait_scatter(b):
        pltpu.make_async_copy(rows[b], acc.at[didx.at[0]], ssem[b]).wait()

    for u in range(NAHEAD):
        gather(u, u)

    def body(t, _):
        for u in range(NBUF):
            c = t * NBUF + u
            wait_gather(u)
            scatter(c, u)
            c2 = c + NAHEAD
            b2 = (u + NAHEAD) % NBUF

            @pl.when(c2 < NCH)
            def _():
                @pl.when(c2 >= NBUF)
                def _():
                    wait_scatter(b2)

                gather(c2, b2)

        return 0

    lax.fori_loop(0, NCH // NBUF, body, 0)
    for u in range(NBUF):
        wait_scatter(u)
    plsc.subcore_barrier()

    def cbody(j, _):
        r0 = sid * RPT + j * CH
        pltpu.sync_copy(acc.at[pl.ds(r0, CH)], rows[0])
        pltpu.sync_copy(rows[0], out_hbm.at[pl.ds(cid * NP + r0, CH)])
        return 0

    lax.fori_loop(0, RPT // CH, cbody, 0)


def _dis_block(degp):
    return lax.rsqrt(1.0 + jnp.sum(degp, axis=0))[:, None]


def _tc1_body(degp_ref, x_ref, w1_ref, g1_ref):
    dis = _dis_block(degp_ref[...])
    h = jnp.dot(x_ref[...], w1_ref[...], preferred_element_type=jnp.float32)
    g1_ref[...] = (h * dis).astype(jnp.bfloat16)


def _tc2_body(accp_ref, g1_ref, degp_ref, b1_ref, w2_ref, g2_ref):
    dis = _dis_block(degp_ref[...])
    a = (accp_ref[0].astype(jnp.float32) + accp_ref[1].astype(jnp.float32)
         + g1_ref[...].astype(jnp.float32))
    h1 = jnp.maximum(a * dis + b1_ref[...], 0.0)
    g2_ref[...] = (jnp.dot(h1, w2_ref[...], preferred_element_type=jnp.float32)
                   * dis).astype(jnp.bfloat16)


def _tc3_body(accp_ref, g2_ref, degp_ref, b2_ref, batch_ref, wct_ref, bc_ref,
              out_ref, s_acc, c_acc):
    i = pl.program_id(0)

    @pl.when(i == 0)
    def _():
        s_acc[...] = jnp.zeros((B, H), jnp.float32)
        c_acc[...] = jnp.zeros((B, H), jnp.float32)

    dis = _dis_block(degp_ref[...])
    a = (accp_ref[0].astype(jnp.float32) + accp_ref[1].astype(jnp.float32)
         + g2_ref[...].astype(jnp.float32))
    h = jnp.maximum(a * dis + b2_ref[...], 0.0)

    bt = batch_ref[0, 0, :]
    classes = lax.broadcasted_iota(jnp.int32, (B, R), 0)
    onehot = (classes == bt[None, :]).astype(jnp.float32)
    s_acc[...] += jnp.dot(onehot, h, preferred_element_type=jnp.float32)
    c_acc[...] += jnp.broadcast_to(
        jnp.sum(onehot, axis=1, keepdims=True), (B, H))

    @pl.when(i == GRID - 1)
    def _():
        pooled = s_acc[...] / jnp.maximum(c_acc[...], 1.0)
        logits = jnp.sum(pooled * wct_ref[...], axis=1, keepdims=True) + bc_ref[0]
        out_ref[...] = jax.nn.sigmoid(logits)


def _tc1(degp, xp, W1):
    return pl.pallas_call(
        _tc1_body,
        grid=(GRID,),
        in_specs=[
            pl.BlockSpec((NW, R), lambda i: (0, i)),
            pl.BlockSpec((R, D), lambda i: (i, 0)),
            pl.BlockSpec((D, H), lambda i: (0, 0)),
        ],
        out_specs=pl.BlockSpec((R, H), lambda i: (i, 0)),
        out_shape=jax.ShapeDtypeStruct((NP, H), jnp.bfloat16),
    )(degp, xp, W1)


def _tc2(accp, g1, degp, b1r, W2):
    return pl.pallas_call(
        _tc2_body,
        grid=(GRID,),
        in_specs=[
            pl.BlockSpec((2, R, H), lambda i: (0, i, 0)),
            pl.BlockSpec((R, H), lambda i: (i, 0)),
            pl.BlockSpec((NW, R), lambda i: (0, i)),
            pl.BlockSpec((1, H), lambda i: (0, 0)),
            pl.BlockSpec((H, H), lambda i: (0, 0)),
        ],
        out_specs=pl.BlockSpec((R, H), lambda i: (i, 0)),
        out_shape=jax.ShapeDtypeStruct((NP, H), jnp.bfloat16),
    )(accp, g1, degp, b1r, W2)


def _tc3(accp, g2, degp, b2r, batch3d, WcT, bc):
    return pl.pallas_call(
        _tc3_body,
        grid=(GRID,),
        in_specs=[
            pl.BlockSpec((2, R, H), lambda i: (0, i, 0)),
            pl.BlockSpec((R, H), lambda i: (i, 0)),
            pl.BlockSpec((NW, R), lambda i: (0, i)),
            pl.BlockSpec((1, H), lambda i: (0, 0)),
            pl.BlockSpec((1, 1, R), lambda i: (i, 0, 0)),
            pl.BlockSpec((1, H), lambda i: (0, 0)),
            pl.BlockSpec(memory_space=pltpu.SMEM),
        ],
        out_specs=pl.BlockSpec((B, 1), lambda i: (0, 0)),
        out_shape=jax.ShapeDtypeStruct((B, 1), jnp.float32),
        scratch_shapes=[
            pltpu.VMEM((B, H), jnp.float32),
            pltpu.VMEM((B, H), jnp.float32),
        ],
    )(accp, g2, degp, b2r, batch3d, WcT, bc)


def kernel(x, edge_index, batch, W1, b1, W2, b2, Wc, bc):
    xp = jnp.pad(x, ((0, NP - N), (0, 0)))
    pad_idx = jnp.full((EP - E,), NP - 1, jnp.int32)
    src = jnp.concatenate([edge_index[0], pad_idx])
    dst = jnp.concatenate([edge_index[1], pad_idx])
    src2d = src.reshape(EP // CH, CH)
    dst2d = dst.reshape(EP // CH, CH)
    batch3d = jnp.pad(batch, (0, NP - N), constant_values=B).reshape(GRID, 1, R)
    zrows = jnp.zeros((NP, H), jnp.bfloat16)
    b1r = b1.reshape(1, H)
    b2r = b2.reshape(1, H)
    WcT = Wc.reshape(1, H)

    degp = _deg(dst)
    g1 = _tc1(degp, xp, W1)
    accp1 = _agg(g1, src2d, dst2d, zrows).reshape(2, NP, H)
    g2 = _tc2(accp1, g1, degp, b1r, W2)
    accp2 = _agg(g2, src2d, dst2d, zrows).reshape(2, NP, H)
    out2d = _tc3(accp2, g2, degp, b2r, batch3d, WcT, bc)
    return out2d.reshape(B)
